# Initial kernel scaffold; baseline (speedup 1.0000x reference)
import jax
import jax.numpy as jnp
from jax.experimental import pallas as pl


def kernel(x, Wq, bq, Wk, bk, Wv, bv, Wo, bo):
    raise NotImplementedError("write your pallas kernel here")


# speedup vs baseline: 3.6983x; 3.6983x over previous
import math

import jax
import jax.numpy as jnp
import numpy as np
from jax.experimental import pallas as pl
from jax.experimental.pallas import tpu as pltpu

_B, _L, _D, _H = 4, 2048, 1024, 16
_DK = _D // _H
_TOPK = int(math.log(_L))
_NF = _L // 2
_F2 = 2 * _NF + 8
_PREC = jax.lax.Precision.DEFAULT
_INTERPRET = False


def _dft_mats():
    l = np.arange(_L)
    f = np.arange(_NF)
    ang = 2.0 * np.pi * np.outer(f, l) / _L
    cos = np.cos(ang)
    sin = np.sin(ang)
    alt = np.where(l % 2 == 0, 1.0, -1.0)[None, :]
    mf = np.concatenate([cos, sin, alt, np.zeros((7, _L))], axis=0)
    w = np.full((_NF,), 2.0)
    w[0] = 1.0
    minv_c = (w[None, :] / _L) * cos.T
    minv_s = -(w[None, :] / _L) * sin.T
    return (
        jnp.asarray(mf, jnp.float32),
        jnp.asarray(minv_c, jnp.float32),
        jnp.asarray(minv_s, jnp.float32),
    )


def _dot(a, b, dn=(((1,), (0,)), ((), ()))):
    return jax.lax.dot_general(
        a, b, dn, precision=_PREC, preferred_element_type=jnp.float32
    )


def _p1_body(x_ref, wq_ref, wk_ref, wv_ref, wo_ref, bq_ref, bk_ref, bv_ref,
             q_ref, k_ref, u_ref):
    x = x_ref[...]
    dn = (((1,), (1,)), ((), ()))
    q_ref[...] = _dot(x, wq_ref[...], dn) + bq_ref[...]
    k_ref[...] = _dot(x, wk_ref[...], dn) + bk_ref[...]
    v = _dot(x, wv_ref[...], dn) + bv_ref[...]
    u_ref[...] = _dot(v, wo_ref[...], dn)


def _p1(x2d, Wq, Wk, Wv, Wo, bq, bk, bv):
    tm = 512
    nrow = _B * _L
    grid = (nrow // tm,)
    wspec = pl.BlockSpec((_D, _D), lambda m: (0, 0))
    bspec = pl.BlockSpec((1, _D), lambda m: (0, 0))
    rspec = pl.BlockSpec((tm, _D), lambda m: (m, 0))
    out_sd = jax.ShapeDtypeStruct((nrow, _D), jnp.float32)
    return pl.pallas_call(
        _p1_body,
        grid=grid,
        in_specs=[rspec, wspec, wspec, wspec, wspec, bspec, bspec, bspec],
        out_specs=[rspec, rspec, rspec],
        out_shape=[out_sd, out_sd, out_sd],
        interpret=_INTERPRET,
    )(x2d, Wq, Wk, Wv, Wo, bq.reshape(1, _D), bk.reshape(1, _D),
      bv.reshape(1, _D))


def _p2a_body(mf_ref, q_ref, k_ref, gq_ref, gk_ref):
    mf = mf_ref[...]
    gq_ref[0] = _dot(mf, q_ref[0])
    gk_ref[0] = _dot(mf, k_ref[0])


def _p2a(mf, q, k):
    dt = 512
    grid = (_B, _D // dt)
    mspec = pl.BlockSpec((_F2, _L), lambda b, d: (0, 0))
    ispec = pl.BlockSpec((1, _L, dt), lambda b, d: (b, 0, d))
    ospec = pl.BlockSpec((1, _F2, dt), lambda b, d: (b, 0, d))
    osd = jax.ShapeDtypeStruct((_B, _F2, _D), jnp.float32)
    return pl.pallas_call(
        _p2a_body,
        grid=grid,
        in_specs=[mspec, ispec, ispec],
        out_specs=[ospec, ospec],
        out_shape=[osd, osd],
        interpret=_INTERPRET,
    )(mf, q, k)


def _p2b_body(mc_ref, ms_ref, gq_ref, gk_ref, corr_ref, msum_ref):
    d = pl.program_id(1)
    gq = gq_ref[0]
    gk = gk_ref[0]
    qc, qs, qa = gq[:_NF], gq[_NF:2 * _NF], gq[2 * _NF:2 * _NF + 1]
    kc, ks, ka = gk[:_NF], gk[_NF:2 * _NF], gk[2 * _NF:2 * _NF + 1]
    rre = qc * kc + qs * ks
    rim = qc * ks - qs * kc
    ra = qa * ka * (1.0 / _L)
    corr = _dot(mc_ref[...], rre) + _dot(ms_ref[...], rim)
    alt = 1.0 - 2.0 * (
        jax.lax.broadcasted_iota(jnp.int32, corr.shape, 0) % 2
    ).astype(jnp.float32)
    corr = corr + alt * ra
    corr_ref[0] = corr
    part = jnp.zeros((_L, 128), jnp.float32)
    for j in range(corr.shape[1] // 128):
        part = part + corr[:, j * 128:(j + 1) * 128]

    @pl.when(d == 0)
    def _():
        msum_ref[0] = part

    @pl.when(d != 0)
    def _():
        msum_ref[0] = msum_ref[0] + part


def _p2b(minv_c, minv_s, gq, gk):
    dt = 512
    grid = (_B, _D // dt)
    mspec = pl.BlockSpec((_L, _NF), lambda b, d: (0, 0))
    gspec = pl.BlockSpec((1, _F2, dt), lambda b, d: (b, 0, d))
    cspec = pl.BlockSpec((1, _L, dt), lambda b, d: (b, 0, d))
    sspec = pl.BlockSpec((1, _L, 128), lambda b, d: (b, 0, 0))
    return pl.pallas_call(
        _p2b_body,
        grid=grid,
        in_specs=[mspec, mspec, gspec, gspec],
        out_specs=[cspec, sspec],
        out_shape=[
            jax.ShapeDtypeStruct((_B, _L, _D), jnp.float32),
            jax.ShapeDtypeStruct((_B, _L, 128), jnp.float32),
        ],
        interpret=_INTERPRET,
    )(minv_c, minv_s, gq, gk)


def _p3_body(msum_ref, idx_ref, w_ref):
    ms = msum_ref[...]
    msl = jnp.sum(ms, axis=2, keepdims=True) * (1.0 / _D)
    mv = jnp.mean(msl, axis=0, keepdims=True)
    iota = jax.lax.broadcasted_iota(jnp.int32, (1, _L, 1), 1)
    idxs = []
    wcols = []
    for _ in range(_TOPK):
        m = jnp.max(mv)
        ii = jnp.min(jnp.where(mv == m, iota, _L))
        sel = iota == ii
        wcols.append(jnp.sum(jnp.where(sel, msl, 0.0), axis=1))
        idxs.append(ii)
        mv = jnp.where(sel, -jnp.inf, mv)
    wmat = jnp.concatenate(wcols, axis=1)
    wmax = jnp.max(wmat, axis=1, keepdims=True)
    e = jnp.exp(wmat - wmax)
    w = e / jnp.sum(e, axis=1, keepdims=True)
    lane8 = jax.lax.broadcasted_iota(jnp.int32, (8, 128), 1)
    row8 = jax.lax.broadcasted_iota(jnp.int32, (8, 128), 0)
    idx_out = jnp.zeros((8, 128), jnp.int32)
    w_out = jnp.zeros((8, 128), jnp.float32)
    for i in range(_TOPK):
        idx_out = jnp.where(lane8 == i, idxs[i], idx_out)
        for b in range(_B):
            w_out = jnp.where((lane8 == i) & (row8 == b), w[b, i], w_out)
    idx_ref[...] = idx_out
    w_ref[...] = w_out


def _p3(msum):
    return pl.pallas_call(
        _p3_body,
        out_shape=[
            jax.ShapeDtypeStruct((8, 128), jnp.int32),
            jax.ShapeDtypeStruct((8, 128), jnp.float32),
        ],
        interpret=_INTERPRET,
    )(msum)


def _p4_body(idx_ref, w_ref, u_ref, bo_ref, out_ref):
    b = pl.program_id(0)
    row = jax.lax.broadcasted_iota(jnp.int32, (_L, _L), 0)
    col = jax.lax.broadcasted_iota(jnp.int32, (_L, _L), 1)
    diff = col - row + 2 * _L
    p = jnp.zeros((_L, _L), jnp.float32)
    for i in range(_TOPK):
        s = idx_ref[i]
        wbi = w_ref[b * 8 + i]
        p = p + jnp.where((diff - s) % _L == 0, wbi, 0.0)
    out_ref[0] = _dot(p, u_ref[0]) + bo_ref[...]


def _p4(idx_flat, w_flat, u, bo):
    gspec = pltpu.PrefetchScalarGridSpec(
        num_scalar_prefetch=2,
        grid=(_B,),
        in_specs=[
            pl.BlockSpec((1, _L, _D), lambda b, *_: (b, 0, 0)),
            pl.BlockSpec((1, _D), lambda b, *_: (0, 0)),
        ],
        out_specs=pl.BlockSpec((1, _L, _D), lambda b, *_: (b, 0, 0)),
    )
    return pl.pallas_call(
        _p4_body,
        grid_spec=gspec,
        out_shape=jax.ShapeDtypeStruct((_B, _L, _D), jnp.float32),
        interpret=_INTERPRET,
    )(idx_flat, w_flat, u, bo.reshape(1, _D))


def kernel(x, Wq, bq, Wk, bk, Wv, bv, Wo, bo):
    mf, minv_c, minv_s = _dft_mats()
    x2d = x.reshape(_B * _L, _D)
    q2d, k2d, u2d = _p1(x2d, Wq, Wk, Wv, Wo, bq, bk, bv)
    q = q2d.reshape(_B, _L, _D)
    k = k2d.reshape(_B, _L, _D)
    u = u2d.reshape(_B, _L, _D)
    gq, gk = _p2a(mf, q, k)
    corr, msum = _p2b(minv_c, minv_s, gq, gk)
    idx_out, w_out = _p3(msum)
    idx_flat = idx_out[0, :8]
    w_flat = w_out[:4, :8].reshape(32)
    out = _p4(idx_flat, w_flat, u, bo)
    attn = corr.reshape(_B, _L, _H, _DK)
    return out, attn


# speedup vs baseline: 6.2643x; 1.6938x over previous
import math

import jax
import jax.numpy as jnp
import numpy as np
from jax.experimental import pallas as pl
from jax.experimental.pallas import tpu as pltpu

_B, _L, _D, _H = 4, 2048, 1024, 16
_DK = _D // _H
_TOPK = int(math.log(_L))
_NF = _L // 2
_F2 = 2 * _NF + 8
_PREC = jax.lax.Precision.DEFAULT
_INTERPRET = False


def _dft_mats():
    l = np.arange(_L)
    f = np.arange(_NF)
    ang = 2.0 * np.pi * np.outer(f, l) / _L
    cos = np.cos(ang)
    sin = np.sin(ang)
    alt = np.where(l % 2 == 0, 1.0, -1.0)[None, :]
    mf = np.concatenate([cos, sin, alt, np.zeros((7, _L))], axis=0)
    w = np.full((_NF,), 2.0)
    w[0] = 1.0
    minv_c = (w[None, :] / _L) * cos.T
    minv_s = -(w[None, :] / _L) * sin.T
    return (
        jnp.asarray(mf, jnp.float32),
        jnp.asarray(minv_c, jnp.bfloat16),
        jnp.asarray(minv_s, jnp.bfloat16),
    )


def _dot(a, b, dn=(((1,), (0,)), ((), ()))):
    return jax.lax.dot_general(
        a, b, dn, precision=_PREC, preferred_element_type=jnp.float32
    )


def _p1_body(x_ref, wq_ref, wk_ref, wv_ref, wo_ref, bq_ref, bk_ref, bv_ref,
             q_ref, k_ref, u_ref):
    x = x_ref[...]
    dn = (((1,), (1,)), ((), ()))
    q_ref[...] = _dot(x, wq_ref[...], dn) + bq_ref[...]
    k_ref[...] = _dot(x, wk_ref[...], dn) + bk_ref[...]
    v = _dot(x, wv_ref[...], dn) + bv_ref[...]
    u_ref[...] = _dot(v, wo_ref[...], dn)


def _p1(x2d, Wq, Wk, Wv, Wo, bq, bk, bv):
    tm = 512
    nrow = _B * _L
    grid = (nrow // tm,)
    wspec = pl.BlockSpec((_D, _D), lambda m: (0, 0))
    bspec = pl.BlockSpec((1, _D), lambda m: (0, 0))
    rspec = pl.BlockSpec((tm, _D), lambda m: (m, 0))
    out_sd = jax.ShapeDtypeStruct((nrow, _D), jnp.float32)
    return pl.pallas_call(
        _p1_body,
        grid=grid,
        in_specs=[rspec, wspec, wspec, wspec, wspec, bspec, bspec, bspec],
        out_specs=[rspec, rspec, rspec],
        out_shape=[out_sd, out_sd, out_sd],
        interpret=_INTERPRET,
    )(x2d, Wq, Wk, Wv, Wo, bq.reshape(1, _D), bk.reshape(1, _D),
      bv.reshape(1, _D))


def _p2a_body(mf_ref, q_ref, k_ref, gq_ref, gk_ref):
    mf = mf_ref[...]
    gq_ref[0] = _dot(mf, q_ref[0])
    gk_ref[0] = _dot(mf, k_ref[0])


def _p2a(mf, q, k):
    dt = 512
    grid = (_B, _D // dt)
    mspec = pl.BlockSpec((_F2, _L), lambda b, d: (0, 0))
    ispec = pl.BlockSpec((1, _L, dt), lambda b, d: (b, 0, d))
    ospec = pl.BlockSpec((1, _F2, dt), lambda b, d: (b, 0, d))
    osd = jax.ShapeDtypeStruct((_B, _F2, _D), jnp.float32)
    return pl.pallas_call(
        _p2a_body,
        grid=grid,
        in_specs=[mspec, ispec, ispec],
        out_specs=[ospec, ospec],
        out_shape=[osd, osd],
        interpret=_INTERPRET,
    )(mf, q, k)


def _p2b_body(mc_ref, ms_ref, gq_ref, gk_ref, corr_ref, msum_ref):
    d = pl.program_id(1)
    gq = gq_ref[0]
    gk = gk_ref[0]
    qc, qs, qa = gq[:_NF], gq[_NF:2 * _NF], gq[2 * _NF:2 * _NF + 1]
    kc, ks, ka = gk[:_NF], gk[_NF:2 * _NF], gk[2 * _NF:2 * _NF + 1]
    rre = (qc * kc + qs * ks).astype(jnp.bfloat16)
    rim = (qc * ks - qs * kc).astype(jnp.bfloat16)
    ra = qa * ka * (1.0 / _L)
    corr = _dot(mc_ref[...], rre) + _dot(ms_ref[...], rim)
    alt = 1.0 - 2.0 * (
        jax.lax.broadcasted_iota(jnp.int32, corr.shape, 0) % 2
    ).astype(jnp.float32)
    corr = corr + alt * ra
    corr_ref[0] = corr
    part = jnp.zeros((_L, 128), jnp.float32)
    for j in range(corr.shape[1] // 128):
        part = part + corr[:, j * 128:(j + 1) * 128]

    @pl.when(d == 0)
    def _():
        msum_ref[0] = part

    @pl.when(d != 0)
    def _():
        msum_ref[0] = msum_ref[0] + part


def _p2b(minv_c, minv_s, gq, gk):
    dt = 512
    grid = (_B, _D // dt)
    mspec = pl.BlockSpec((_L, _NF), lambda b, d: (0, 0))
    gspec = pl.BlockSpec((1, _F2, dt), lambda b, d: (b, 0, d))
    cspec = pl.BlockSpec((1, _L, dt), lambda b, d: (b, 0, d))
    sspec = pl.BlockSpec((1, _L, 128), lambda b, d: (b, 0, 0))
    return pl.pallas_call(
        _p2b_body,
        grid=grid,
        in_specs=[mspec, mspec, gspec, gspec],
        out_specs=[cspec, sspec],
        out_shape=[
            jax.ShapeDtypeStruct((_B, _L, _D), jnp.float32),
            jax.ShapeDtypeStruct((_B, _L, 128), jnp.float32),
        ],
        interpret=_INTERPRET,
    )(minv_c, minv_s, gq, gk)


def _p3_body(msum_ref, idx_ref, w_ref):
    ms = msum_ref[...]
    msl = jnp.sum(ms, axis=2, keepdims=True) * (1.0 / _D)
    mv = jnp.mean(msl, axis=0, keepdims=True)
    iota = jax.lax.broadcasted_iota(jnp.int32, (1, _L, 1), 1)
    idxs = []
    wcols = []
    for _ in range(_TOPK):
        m = jnp.max(mv)
        ii = jnp.min(jnp.where(mv == m, iota, _L))
        sel = iota == ii
        wcols.append(jnp.sum(jnp.where(sel, msl, 0.0), axis=1))
        idxs.append(ii)
        mv = jnp.where(sel, -jnp.inf, mv)
    wmat = jnp.concatenate(wcols, axis=1)
    wmax = jnp.max(wmat, axis=1, keepdims=True)
    e = jnp.exp(wmat - wmax)
    w = e / jnp.sum(e, axis=1, keepdims=True)
    lane8 = jax.lax.broadcasted_iota(jnp.int32, (8, 128), 1)
    row8 = jax.lax.broadcasted_iota(jnp.int32, (8, 128), 0)
    idx_out = jnp.zeros((8, 128), jnp.int32)
    w_out = jnp.zeros((8, 128), jnp.float32)
    for i in range(_TOPK):
        idx_out = jnp.where(lane8 == i, idxs[i], idx_out)
        for b in range(_B):
            w_out = jnp.where((lane8 == i) & (row8 == b), w[b, i], w_out)
    idx_ref[...] = idx_out
    w_ref[...] = w_out


def _p3(msum):
    return pl.pallas_call(
        _p3_body,
        out_shape=[
            jax.ShapeDtypeStruct((8, 128), jnp.int32),
            jax.ShapeDtypeStruct((8, 128), jnp.float32),
        ],
        interpret=_INTERPRET,
    )(msum)


def _p4_body(idx_ref, w_ref, u_ref, bo_ref, out_ref):
    b = pl.program_id(0)
    row = jax.lax.broadcasted_iota(jnp.int32, (_L, _L), 0)
    col = jax.lax.broadcasted_iota(jnp.int32, (_L, _L), 1)
    diff = col - row + 2 * _L
    dm = diff % _L
    p = jnp.zeros((_L, _L), jnp.float32)
    for i in range(_TOPK):
        s = idx_ref[i]
        wbi = w_ref[b * 8 + i]
        p = jnp.where(dm == s, wbi, p)
    out_ref[0] = _dot(p.astype(jnp.bfloat16), u_ref[0].astype(jnp.bfloat16)) + bo_ref[...]


def _p4(idx_flat, w_flat, u, bo):
    gspec = pltpu.PrefetchScalarGridSpec(
        num_scalar_prefetch=2,
        grid=(_B,),
        in_specs=[
            pl.BlockSpec((1, _L, _D), lambda b, *_: (b, 0, 0)),
            pl.BlockSpec((1, _D), lambda b, *_: (0, 0)),
        ],
        out_specs=pl.BlockSpec((1, _L, _D), lambda b, *_: (b, 0, 0)),
    )
    return pl.pallas_call(
        _p4_body,
        grid_spec=gspec,
        out_shape=jax.ShapeDtypeStruct((_B, _L, _D), jnp.float32),
        interpret=_INTERPRET,
    )(idx_flat, w_flat, u, bo.reshape(1, _D))


def kernel(x, Wq, bq, Wk, bk, Wv, bv, Wo, bo):
    mf, minv_c, minv_s = _dft_mats()
    x2d = x.reshape(_B * _L, _D)
    q2d, k2d, u2d = _p1(x2d, Wq, Wk, Wv, Wo, bq, bk, bv)
    q = q2d.reshape(_B, _L, _D)
    k = k2d.reshape(_B, _L, _D)
    u = u2d.reshape(_B, _L, _D)
    gq, gk = _p2a(mf, q, k)
    corr, msum = _p2b(minv_c, minv_s, gq, gk)
    idx_out, w_out = _p3(msum)
    idx_flat = idx_out[0, :8]
    w_flat = w_out[:4, :8].reshape(32)
    out = _p4(idx_flat, w_flat, u, bo)
    attn = corr.reshape(_B, _L, _H, _DK)
    return out, attn
